# baseline (speedup 1.0000x reference)
import functools

import jax
import jax.numpy as jnp
from jax.experimental import pallas as pl
from jax.experimental.pallas import tpu as pltpu

_LAT = 256
_NH = 4
_HD = _LAT // _NH


def _qkv_kernel(e_ref, q_ref, k_ref, v_ref, qo_ref, ko_ref, vo_ref):
    x = e_ref[...]
    qo_ref[...] = jnp.dot(x, q_ref[...], preferred_element_type=jnp.float32)
    ko_ref[...] = jnp.dot(x, k_ref[...], preferred_element_type=jnp.float32)
    vo_ref[...] = jnp.dot(x, v_ref[...], preferred_element_type=jnp.float32)


def _head_mask():
    d_ids = jax.lax.broadcasted_iota(jnp.int32, (_LAT, _NH), 0) // _HD
    h_ids = jax.lax.broadcasted_iota(jnp.int32, (_LAT, _NH), 1)
    return (d_ids == h_ids).astype(jnp.float32)


def _pass1_kernel(rows_ref, cols_ref, qn_ref, kn_ref, exp_ref, norm_ref, *, blk):
    i = pl.program_id(0)

    @pl.when(i == 0)
    def _init():
        norm_ref[...] = jnp.zeros_like(norm_ref)

    mask = _head_mask()

    def body(e, carry):
        r = rows_ref[0, 0, e]
        c = cols_ref[0, 0, e]
        q = qn_ref[pl.ds(r, 1), :]
        k = kn_ref[pl.ds(c, 1), :]
        att = jnp.dot(q * k, mask, preferred_element_type=jnp.float32)
        att = jnp.clip(att, -10.0, 10.0)
        ea = jnp.exp(att)
        exp_ref[pl.ds(e, 1), :] = ea
        norm_ref[pl.ds(r, 1), :] = norm_ref[pl.ds(r, 1), :] + ea
        return carry

    jax.lax.fori_loop(0, blk, body, 0)


def _pass2_kernel(rows_ref, cols_ref, exp_ref, norm_ref, vn_ref, att_ref,
                  out_ref, *, blk):
    i = pl.program_id(0)

    @pl.when(i == 0)
    def _init():
        out_ref[...] = jnp.zeros_like(out_ref)

    mask_t = _head_mask().T

    def body(e, carry):
        r = rows_ref[0, 0, e]
        c = cols_ref[0, 0, e]
        ea = exp_ref[pl.ds(e, 1), :]
        nrm = norm_ref[pl.ds(r, 1), :]
        a = ea / (nrm + 1e-8)
        att_ref[pl.ds(e, 1), :] = a
        scale = jnp.dot(a, mask_t, preferred_element_type=jnp.float32)
        v = vn_ref[pl.ds(c, 1), :]
        out_ref[pl.ds(r, 1), :] = out_ref[pl.ds(r, 1), :] + scale * v
        return carry

    jax.lax.fori_loop(0, blk, body, 0)


def kernel(adj_indices, embeds, qTrans, kTrans, vTrans):
    n_nodes, lat = embeds.shape
    n_edges = adj_indices.shape[1]
    rows = adj_indices[0, :]
    cols = adj_indices[1, :]

    node_blk = 1000 if n_nodes % 1000 == 0 else n_nodes
    n_grid = n_nodes // node_blk
    qn, kn, vn = pl.pallas_call(
        _qkv_kernel,
        grid=(n_grid,),
        in_specs=[
            pl.BlockSpec((node_blk, lat), lambda i: (i, 0)),
            pl.BlockSpec((lat, lat), lambda i: (0, 0)),
            pl.BlockSpec((lat, lat), lambda i: (0, 0)),
            pl.BlockSpec((lat, lat), lambda i: (0, 0)),
        ],
        out_specs=[
            pl.BlockSpec((node_blk, lat), lambda i: (i, 0)),
            pl.BlockSpec((node_blk, lat), lambda i: (i, 0)),
            pl.BlockSpec((node_blk, lat), lambda i: (i, 0)),
        ],
        out_shape=[jax.ShapeDtypeStruct((n_nodes, lat), jnp.float32)] * 3,
    )(embeds, qTrans, kTrans, vTrans)

    edge_blk = 3200 if n_edges % 3200 == 0 else n_edges
    n_eblk = n_edges // edge_blk
    rows2 = rows.reshape(n_eblk, 1, edge_blk)
    cols2 = cols.reshape(n_eblk, 1, edge_blk)

    idx_spec = pl.BlockSpec((1, 1, edge_blk), lambda i: (i, 0, 0),
                            memory_space=pltpu.SMEM)
    full_tab = lambda shape: pl.BlockSpec(shape, lambda i: (0, 0))

    exp_att, norm = pl.pallas_call(
        functools.partial(_pass1_kernel, blk=edge_blk),
        grid=(n_eblk,),
        in_specs=[
            idx_spec,
            idx_spec,
            full_tab((n_nodes, lat)),
            full_tab((n_nodes, lat)),
        ],
        out_specs=[
            pl.BlockSpec((edge_blk, _NH), lambda i: (i, 0)),
            pl.BlockSpec((n_nodes, _NH), lambda i: (0, 0)),
        ],
        out_shape=[
            jax.ShapeDtypeStruct((n_edges, _NH), jnp.float32),
            jax.ShapeDtypeStruct((n_nodes, _NH), jnp.float32),
        ],
    )(rows2, cols2, qn, kn)

    att, res = pl.pallas_call(
        functools.partial(_pass2_kernel, blk=edge_blk),
        grid=(n_eblk,),
        in_specs=[
            idx_spec,
            idx_spec,
            pl.BlockSpec((edge_blk, _NH), lambda i: (i, 0)),
            full_tab((n_nodes, _NH)),
            full_tab((n_nodes, lat)),
        ],
        out_specs=[
            pl.BlockSpec((edge_blk, _NH), lambda i: (i, 0)),
            pl.BlockSpec((n_nodes, lat), lambda i: (0, 0)),
        ],
        out_shape=[
            jax.ShapeDtypeStruct((n_edges, _NH), jnp.float32),
            jax.ShapeDtypeStruct((n_nodes, lat), jnp.float32),
        ],
    )(rows2, cols2, exp_att, norm, vn)

    return res, att


# speedup vs baseline: 5.1769x; 5.1769x over previous
import functools

import jax
import jax.numpy as jnp
from jax.experimental import pallas as pl
from jax.experimental.pallas import tpu as pltpu

_LAT = 256
_NH = 4
_HD = _LAT // _NH


def _qkv_kernel(e_ref, q_ref, k_ref, v_ref, qo_ref, ko_ref, vo_ref):
    x = e_ref[...]
    qo_ref[...] = jnp.dot(x, q_ref[...], preferred_element_type=jnp.float32)
    ko_ref[...] = jnp.dot(x, k_ref[...], preferred_element_type=jnp.float32)
    vo_ref[...] = jnp.dot(x, v_ref[...], preferred_element_type=jnp.float32)


def _head_mask():
    d_ids = jax.lax.broadcasted_iota(jnp.int32, (_LAT, _NH), 0) // _HD
    h_ids = jax.lax.broadcasted_iota(jnp.int32, (_LAT, _NH), 1)
    return (d_ids == h_ids).astype(jnp.float32)


def _pass1_kernel(rows_ref, cols_ref, qn_ref, kn_ref, exp_ref, norm_ref, *, blk):
    i = pl.program_id(0)

    @pl.when(i == 0)
    def _init():
        norm_ref[...] = jnp.zeros_like(norm_ref)

    mask = _head_mask()

    def body(e, carry):
        r = rows_ref[0, 0, e]
        c = cols_ref[0, 0, e]
        q = qn_ref[pl.ds(r, 1), :]
        k = kn_ref[pl.ds(c, 1), :]
        att = jnp.dot(q * k, mask, preferred_element_type=jnp.float32)
        att = jnp.clip(att, -10.0, 10.0)
        ea = jnp.exp(att)
        exp_ref[pl.ds(e, 1), :] = ea
        norm_ref[pl.ds(r, 1), :] = norm_ref[pl.ds(r, 1), :] + ea
        return carry

    jax.lax.fori_loop(0, blk, body, 0, unroll=8)


def _pass2_kernel(rows_ref, cols_ref, exp_ref, norm_ref, vn_ref, att_ref,
                  out_ref, *, blk):
    i = pl.program_id(0)

    @pl.when(i == 0)
    def _init():
        out_ref[...] = jnp.zeros_like(out_ref)

    mask_t = _head_mask().T

    def body(e, carry):
        r = rows_ref[0, 0, e]
        c = cols_ref[0, 0, e]
        ea = exp_ref[pl.ds(e, 1), :]
        nrm = norm_ref[pl.ds(r, 1), :]
        a = ea / (nrm + 1e-8)
        att_ref[pl.ds(e, 1), :] = a
        scale = jnp.dot(a, mask_t, preferred_element_type=jnp.float32)
        v = vn_ref[pl.ds(c, 1), :]
        out_ref[pl.ds(r, 1), :] = out_ref[pl.ds(r, 1), :] + scale * v
        return carry

    jax.lax.fori_loop(0, blk, body, 0, unroll=8)


def kernel(adj_indices, embeds, qTrans, kTrans, vTrans):
    n_nodes, lat = embeds.shape
    n_edges = adj_indices.shape[1]
    rows = adj_indices[0, :]
    cols = adj_indices[1, :]

    node_blk = 1000 if n_nodes % 1000 == 0 else n_nodes
    n_grid = n_nodes // node_blk
    qn, kn, vn = pl.pallas_call(
        _qkv_kernel,
        grid=(n_grid,),
        in_specs=[
            pl.BlockSpec((node_blk, lat), lambda i: (i, 0)),
            pl.BlockSpec((lat, lat), lambda i: (0, 0)),
            pl.BlockSpec((lat, lat), lambda i: (0, 0)),
            pl.BlockSpec((lat, lat), lambda i: (0, 0)),
        ],
        out_specs=[
            pl.BlockSpec((node_blk, lat), lambda i: (i, 0)),
            pl.BlockSpec((node_blk, lat), lambda i: (i, 0)),
            pl.BlockSpec((node_blk, lat), lambda i: (i, 0)),
        ],
        out_shape=[jax.ShapeDtypeStruct((n_nodes, lat), jnp.float32)] * 3,
    )(embeds, qTrans, kTrans, vTrans)

    edge_blk = 3200 if n_edges % 3200 == 0 else n_edges
    n_eblk = n_edges // edge_blk
    rows2 = rows.reshape(n_eblk, 1, edge_blk)
    cols2 = cols.reshape(n_eblk, 1, edge_blk)

    idx_spec = pl.BlockSpec((1, 1, edge_blk), lambda i: (i, 0, 0),
                            memory_space=pltpu.SMEM)
    full_tab = lambda shape: pl.BlockSpec(shape, lambda i: (0, 0))

    exp_att, norm = pl.pallas_call(
        functools.partial(_pass1_kernel, blk=edge_blk),
        grid=(n_eblk,),
        in_specs=[
            idx_spec,
            idx_spec,
            full_tab((n_nodes, lat)),
            full_tab((n_nodes, lat)),
        ],
        out_specs=[
            pl.BlockSpec((edge_blk, _NH), lambda i: (i, 0)),
            pl.BlockSpec((n_nodes, _NH), lambda i: (0, 0)),
        ],
        out_shape=[
            jax.ShapeDtypeStruct((n_edges, _NH), jnp.float32),
            jax.ShapeDtypeStruct((n_nodes, _NH), jnp.float32),
        ],
    )(rows2, cols2, qn, kn)

    att, res = pl.pallas_call(
        functools.partial(_pass2_kernel, blk=edge_blk),
        grid=(n_eblk,),
        in_specs=[
            idx_spec,
            idx_spec,
            pl.BlockSpec((edge_blk, _NH), lambda i: (i, 0)),
            full_tab((n_nodes, _NH)),
            full_tab((n_nodes, lat)),
        ],
        out_specs=[
            pl.BlockSpec((edge_blk, _NH), lambda i: (i, 0)),
            pl.BlockSpec((n_nodes, lat), lambda i: (0, 0)),
        ],
        out_shape=[
            jax.ShapeDtypeStruct((n_edges, _NH), jnp.float32),
            jax.ShapeDtypeStruct((n_nodes, lat), jnp.float32),
        ],
    )(rows2, cols2, exp_att, norm, vn)

    return res, att


# speedup vs baseline: 7.1047x; 1.3724x over previous
import functools

import jax
import jax.numpy as jnp
from jax.experimental import pallas as pl
from jax.experimental.pallas import tpu as pltpu

_LAT = 256
_NH = 4
_HD = _LAT // _NH


def _qkv_kernel(e_ref, q_ref, k_ref, v_ref, qo_ref, ko_ref, vo_ref):
    x = e_ref[...]
    qo_ref[...] = jnp.dot(x, q_ref[...], preferred_element_type=jnp.float32)
    ko_ref[...] = jnp.dot(x, k_ref[...], preferred_element_type=jnp.float32)
    vo_ref[...] = jnp.dot(x, v_ref[...], preferred_element_type=jnp.float32)


def _head_mask():
    d_ids = jax.lax.broadcasted_iota(jnp.int32, (_LAT, _NH), 0) // _HD
    h_ids = jax.lax.broadcasted_iota(jnp.int32, (_LAT, _NH), 1)
    return (d_ids == h_ids).astype(jnp.float32)


def _pass1_kernel(rows_ref, cols_ref, qn_ref, kn_ref, exp_ref, norm_ref, *, blk):
    i = pl.program_id(0)

    @pl.when(i == 0)
    def _init():
        norm_ref[...] = jnp.zeros_like(norm_ref)

    mask = _head_mask()

    def body(e, carry):
        r = rows_ref[0, 0, e]
        c = cols_ref[0, 0, e]
        q = qn_ref[pl.ds(r, 1), :]
        k = kn_ref[pl.ds(c, 1), :]
        att = jnp.dot(q * k, mask, preferred_element_type=jnp.float32)
        att = jnp.clip(att, -10.0, 10.0)
        ea = jnp.exp(att)
        exp_ref[pl.ds(e, 1), :] = ea
        norm_ref[pl.ds(r, 1), :] = norm_ref[pl.ds(r, 1), :] + ea
        return carry

    jax.lax.fori_loop(0, blk, body, 0, unroll=16)


def _pass2_kernel(rows_ref, cols_ref, exp_ref, norm_ref, vn_ref, att_ref,
                  out_ref, *, blk):
    i = pl.program_id(0)

    @pl.when(i == 0)
    def _init():
        out_ref[...] = jnp.zeros_like(out_ref)

    mask_t = _head_mask().T

    def body(e, carry):
        r = rows_ref[0, 0, e]
        c = cols_ref[0, 0, e]
        ea = exp_ref[pl.ds(e, 1), :]
        nrm = norm_ref[pl.ds(r, 1), :]
        a = ea / (nrm + 1e-8)
        att_ref[pl.ds(e, 1), :] = a
        scale = jnp.dot(a, mask_t, preferred_element_type=jnp.float32)
        v = vn_ref[pl.ds(c, 1), :]
        out_ref[pl.ds(r, 1), :] = out_ref[pl.ds(r, 1), :] + scale * v
        return carry

    jax.lax.fori_loop(0, blk, body, 0, unroll=16)


def kernel(adj_indices, embeds, qTrans, kTrans, vTrans):
    n_nodes, lat = embeds.shape
    n_edges = adj_indices.shape[1]
    rows = adj_indices[0, :]
    cols = adj_indices[1, :]

    node_blk = 1000 if n_nodes % 1000 == 0 else n_nodes
    n_grid = n_nodes // node_blk
    qn, kn, vn = pl.pallas_call(
        _qkv_kernel,
        grid=(n_grid,),
        in_specs=[
            pl.BlockSpec((node_blk, lat), lambda i: (i, 0)),
            pl.BlockSpec((lat, lat), lambda i: (0, 0)),
            pl.BlockSpec((lat, lat), lambda i: (0, 0)),
            pl.BlockSpec((lat, lat), lambda i: (0, 0)),
        ],
        out_specs=[
            pl.BlockSpec((node_blk, lat), lambda i: (i, 0)),
            pl.BlockSpec((node_blk, lat), lambda i: (i, 0)),
            pl.BlockSpec((node_blk, lat), lambda i: (i, 0)),
        ],
        out_shape=[jax.ShapeDtypeStruct((n_nodes, lat), jnp.float32)] * 3,
    )(embeds, qTrans, kTrans, vTrans)

    edge_blk = 3200 if n_edges % 3200 == 0 else n_edges
    n_eblk = n_edges // edge_blk
    rows2 = rows.reshape(n_eblk, 1, edge_blk)
    cols2 = cols.reshape(n_eblk, 1, edge_blk)

    idx_spec = pl.BlockSpec((1, 1, edge_blk), lambda i: (i, 0, 0),
                            memory_space=pltpu.SMEM)
    full_tab = lambda shape: pl.BlockSpec(shape, lambda i: (0, 0))

    exp_att, norm = pl.pallas_call(
        functools.partial(_pass1_kernel, blk=edge_blk),
        grid=(n_eblk,),
        in_specs=[
            idx_spec,
            idx_spec,
            full_tab((n_nodes, lat)),
            full_tab((n_nodes, lat)),
        ],
        out_specs=[
            pl.BlockSpec((edge_blk, _NH), lambda i: (i, 0)),
            pl.BlockSpec((n_nodes, _NH), lambda i: (0, 0)),
        ],
        out_shape=[
            jax.ShapeDtypeStruct((n_edges, _NH), jnp.float32),
            jax.ShapeDtypeStruct((n_nodes, _NH), jnp.float32),
        ],
    )(rows2, cols2, qn, kn)

    att, res = pl.pallas_call(
        functools.partial(_pass2_kernel, blk=edge_blk),
        grid=(n_eblk,),
        in_specs=[
            idx_spec,
            idx_spec,
            pl.BlockSpec((edge_blk, _NH), lambda i: (i, 0)),
            full_tab((n_nodes, _NH)),
            full_tab((n_nodes, lat)),
        ],
        out_specs=[
            pl.BlockSpec((edge_blk, _NH), lambda i: (i, 0)),
            pl.BlockSpec((n_nodes, lat), lambda i: (0, 0)),
        ],
        out_shape=[
            jax.ShapeDtypeStruct((n_edges, _NH), jnp.float32),
            jax.ShapeDtypeStruct((n_nodes, lat), jnp.float32),
        ],
    )(rows2, cols2, exp_att, norm, vn)

    return res, att


# speedup vs baseline: 7.1092x; 1.0006x over previous
import functools

import jax
import jax.numpy as jnp
from jax.experimental import pallas as pl
from jax.experimental.pallas import tpu as pltpu

_LAT = 256
_NH = 4
_HD = _LAT // _NH


def _qkv_kernel(e_ref, q_ref, k_ref, v_ref, qo_ref, ko_ref, vo_ref):
    x = e_ref[...]
    qo_ref[...] = jnp.dot(x, q_ref[...], preferred_element_type=jnp.float32)
    ko_ref[...] = jnp.dot(x, k_ref[...], preferred_element_type=jnp.float32)
    vo_ref[...] = jnp.dot(x, v_ref[...], preferred_element_type=jnp.float32)


def _head_mask():
    d_ids = jax.lax.broadcasted_iota(jnp.int32, (_LAT, _NH), 0) // _HD
    h_ids = jax.lax.broadcasted_iota(jnp.int32, (_LAT, _NH), 1)
    return (d_ids == h_ids).astype(jnp.float32)


def _pass1_kernel(rows_ref, cols_ref, qn_ref, kn_ref, exp_ref, na_ref, nb_ref,
                  *, blk):
    i = pl.program_id(0)

    @pl.when(i == 0)
    def _init():
        na_ref[...] = jnp.zeros_like(na_ref)
        nb_ref[...] = jnp.zeros_like(nb_ref)

    mask = _head_mask()

    def one_edge(e, norm_ref):
        r = rows_ref[0, 0, e]
        c = cols_ref[0, 0, e]
        q = qn_ref[pl.ds(r, 1), :]
        k = kn_ref[pl.ds(c, 1), :]
        att = jnp.dot(q * k, mask, preferred_element_type=jnp.float32)
        att = jnp.clip(att, -10.0, 10.0)
        ea = jnp.exp(att)
        exp_ref[pl.ds(e, 1), :] = ea
        norm_ref[pl.ds(r, 1), :] = norm_ref[pl.ds(r, 1), :] + ea

    def body(j, carry):
        one_edge(2 * j, na_ref)
        one_edge(2 * j + 1, nb_ref)
        return carry

    jax.lax.fori_loop(0, blk // 2, body, 0, unroll=8)


def _pass2_kernel(rows_ref, cols_ref, exp_ref, na_ref, nb_ref, vn_ref,
                  att_ref, oa_ref, ob_ref, *, blk):
    i = pl.program_id(0)

    @pl.when(i == 0)
    def _init():
        oa_ref[...] = jnp.zeros_like(oa_ref)
        ob_ref[...] = jnp.zeros_like(ob_ref)

    mask_t = _head_mask().T

    def one_edge(e, out_ref):
        r = rows_ref[0, 0, e]
        c = cols_ref[0, 0, e]
        ea = exp_ref[pl.ds(e, 1), :]
        nrm = na_ref[pl.ds(r, 1), :] + nb_ref[pl.ds(r, 1), :]
        a = ea / (nrm + 1e-8)
        att_ref[pl.ds(e, 1), :] = a
        scale = jnp.dot(a, mask_t, preferred_element_type=jnp.float32)
        v = vn_ref[pl.ds(c, 1), :]
        out_ref[pl.ds(r, 1), :] = out_ref[pl.ds(r, 1), :] + scale * v

    def body(j, carry):
        one_edge(2 * j, oa_ref)
        one_edge(2 * j + 1, ob_ref)
        return carry

    jax.lax.fori_loop(0, blk // 2, body, 0, unroll=8)


def _add_kernel(a_ref, b_ref, o_ref):
    o_ref[...] = a_ref[...] + b_ref[...]


def kernel(adj_indices, embeds, qTrans, kTrans, vTrans):
    n_nodes, lat = embeds.shape
    n_edges = adj_indices.shape[1]
    rows = adj_indices[0, :]
    cols = adj_indices[1, :]

    node_blk = 1000 if n_nodes % 1000 == 0 else n_nodes
    n_grid = n_nodes // node_blk
    qn, kn, vn = pl.pallas_call(
        _qkv_kernel,
        grid=(n_grid,),
        in_specs=[
            pl.BlockSpec((node_blk, lat), lambda i: (i, 0)),
            pl.BlockSpec((lat, lat), lambda i: (0, 0)),
            pl.BlockSpec((lat, lat), lambda i: (0, 0)),
            pl.BlockSpec((lat, lat), lambda i: (0, 0)),
        ],
        out_specs=[
            pl.BlockSpec((node_blk, lat), lambda i: (i, 0)),
            pl.BlockSpec((node_blk, lat), lambda i: (i, 0)),
            pl.BlockSpec((node_blk, lat), lambda i: (i, 0)),
        ],
        out_shape=[jax.ShapeDtypeStruct((n_nodes, lat), jnp.float32)] * 3,
    )(embeds, qTrans, kTrans, vTrans)

    edge_blk = 3200 if n_edges % 3200 == 0 else n_edges
    n_eblk = n_edges // edge_blk
    rows2 = rows.reshape(n_eblk, 1, edge_blk)
    cols2 = cols.reshape(n_eblk, 1, edge_blk)

    idx_spec = pl.BlockSpec((1, 1, edge_blk), lambda i: (i, 0, 0),
                            memory_space=pltpu.SMEM)
    full_tab = lambda shape: pl.BlockSpec(shape, lambda i: (0, 0))

    exp_att, norm_a, norm_b = pl.pallas_call(
        functools.partial(_pass1_kernel, blk=edge_blk),
        grid=(n_eblk,),
        in_specs=[
            idx_spec,
            idx_spec,
            full_tab((n_nodes, lat)),
            full_tab((n_nodes, lat)),
        ],
        out_specs=[
            pl.BlockSpec((edge_blk, _NH), lambda i: (i, 0)),
            pl.BlockSpec((n_nodes, _NH), lambda i: (0, 0)),
            pl.BlockSpec((n_nodes, _NH), lambda i: (0, 0)),
        ],
        out_shape=[
            jax.ShapeDtypeStruct((n_edges, _NH), jnp.float32),
            jax.ShapeDtypeStruct((n_nodes, _NH), jnp.float32),
            jax.ShapeDtypeStruct((n_nodes, _NH), jnp.float32),
        ],
    )(rows2, cols2, qn, kn)

    att, out_a, out_b = pl.pallas_call(
        functools.partial(_pass2_kernel, blk=edge_blk),
        grid=(n_eblk,),
        in_specs=[
            idx_spec,
            idx_spec,
            pl.BlockSpec((edge_blk, _NH), lambda i: (i, 0)),
            full_tab((n_nodes, _NH)),
            full_tab((n_nodes, _NH)),
            full_tab((n_nodes, lat)),
        ],
        out_specs=[
            pl.BlockSpec((edge_blk, _NH), lambda i: (i, 0)),
            pl.BlockSpec((n_nodes, lat), lambda i: (0, 0)),
            pl.BlockSpec((n_nodes, lat), lambda i: (0, 0)),
        ],
        out_shape=[
            jax.ShapeDtypeStruct((n_edges, _NH), jnp.float32),
            jax.ShapeDtypeStruct((n_nodes, lat), jnp.float32),
            jax.ShapeDtypeStruct((n_nodes, lat), jnp.float32),
        ],
    )(rows2, cols2, exp_att, norm_a, norm_b, vn)

    res = pl.pallas_call(
        _add_kernel,
        grid=(n_grid,),
        in_specs=[
            pl.BlockSpec((node_blk, lat), lambda i: (i, 0)),
            pl.BlockSpec((node_blk, lat), lambda i: (i, 0)),
        ],
        out_specs=pl.BlockSpec((node_blk, lat), lambda i: (i, 0)),
        out_shape=jax.ShapeDtypeStruct((n_nodes, lat), jnp.float32),
    )(out_a, out_b)

    return res, att
